# Initial kernel scaffold; baseline (speedup 1.0000x reference)
import jax
import jax.numpy as jnp
from jax.experimental import pallas as pl


def kernel(anchor_probs):
    raise NotImplementedError("write your pallas kernel here")


# speedup vs baseline: 1.5822x; 1.5822x over previous
import jax
import jax.numpy as jnp
from jax import lax
from jax.experimental import pallas as pl

_TOP_K = 1000
_CONF = 0.01
_B = 4
_N0 = 4096
_K = 4096
_W = 64
_ROWS = 512
_NSTEP = _N0 // _ROWS
_PADK = 1024


def _reduce_body(x_ref, mv_ref, mi_ref):
    x = x_ref[...]
    m = jnp.max(x, axis=-1)
    iota = lax.broadcasted_iota(jnp.int32, x.shape, 2)
    hit = jnp.where(x == m[..., None], iota, _K)
    mi = jnp.min(hit, axis=-1)
    mv_ref[...] = m[:, None, :]
    mi_ref[...] = mi[:, None, :]


def _partner(x, j, n):
    fwd = jnp.concatenate([x[:, j:], x[:, :j]], axis=1)
    bwd = jnp.concatenate([x[:, n - j:], x[:, :n - j]], axis=1)
    return fwd, bwd


def _topk_body(mv_ref, mi_ref, conf_ref, x0_ref, y0_ref, x1_ref, y1_ref):
    v = mv_ref[...]
    anch = mi_ref[...]
    v = jnp.where(v > _CONF, v, -jnp.inf)
    idx = lax.broadcasted_iota(jnp.int32, (_B, _N0), 1)
    pack = (idx << 12) | anch

    n = _N0
    k = 2
    while k <= n:
        j = k // 2
        while j >= 1:
            bit_lo = (idx & j) == 0
            desc = (idx & k) == 0
            vf, vb = _partner(v, j, n)
            pf, pb = _partner(pack, j, n)
            pv = jnp.where(bit_lo, vf, vb)
            pp = jnp.where(bit_lo, pf, pb)
            self_wins = (v > pv) | ((v == pv) & (pack < pp))
            keep = (self_wins == bit_lo) == desc
            v = jnp.where(keep, v, pv)
            pack = jnp.where(keep, pack, pp)
            j //= 2
        k *= 2

    top = v[:, :_PADK]
    pk = pack[:, :_PADK]
    sidx = pk >> 12
    sanch = pk & (_N0 - 1)
    valid = top > _CONF
    inv = jnp.float32(1.0 / (_W - 1))
    fz = jnp.float32(0.0)
    conf_ref[...] = jnp.where(valid, top, fz)
    x0_ref[...] = jnp.where(valid, (sidx & (_W - 1)).astype(jnp.float32) * inv, fz)
    y0_ref[...] = jnp.where(valid, (sidx >> 6).astype(jnp.float32) * inv, fz)
    x1_ref[...] = jnp.where(valid, (sanch & (_W - 1)).astype(jnp.float32) * inv, fz)
    y1_ref[...] = jnp.where(valid, (sanch >> 6).astype(jnp.float32) * inv, fz)


def kernel(anchor_probs):
    B, N0, K = anchor_probs.shape
    mv3, mi3 = pl.pallas_call(
        _reduce_body,
        grid=(B * _NSTEP,),
        in_specs=[pl.BlockSpec((1, _ROWS, K), lambda g: (g // _NSTEP, g % _NSTEP, 0))],
        out_specs=[
            pl.BlockSpec((1, 1, _ROWS), lambda g: (g, 0, 0)),
            pl.BlockSpec((1, 1, _ROWS), lambda g: (g, 0, 0)),
        ],
        out_shape=[
            jax.ShapeDtypeStruct((B * _NSTEP, 1, _ROWS), jnp.float32),
            jax.ShapeDtypeStruct((B * _NSTEP, 1, _ROWS), jnp.int32),
        ],
    )(anchor_probs)
    maxv = mv3.reshape(B, N0)
    maxi = mi3.reshape(B, N0)

    conf, x0, y0, x1, y1 = pl.pallas_call(
        _topk_body,
        out_shape=[jax.ShapeDtypeStruct((B, _PADK), jnp.float32)] * 5,
    )(maxv, maxi)

    conf = conf[:, :_TOP_K]
    mkpts0 = jnp.stack([x0[:, :_TOP_K], y0[:, :_TOP_K]], axis=-1).reshape(-1, 2)
    mkpts1 = jnp.stack([x1[:, :_TOP_K], y1[:, :_TOP_K]], axis=-1).reshape(-1, 2)
    mconf = conf.reshape(-1)
    b_ids = jnp.broadcast_to(jnp.arange(B)[:, None], (B, _TOP_K)).reshape(-1)
    return (mkpts0, mkpts1, mconf, b_ids)


# speedup vs baseline: 1.7224x; 1.0886x over previous
import jax
import jax.numpy as jnp
from jax import lax
from jax.experimental import pallas as pl

_TOP_K = 1000
_CONF = 0.01
_B = 4
_N0 = 4096
_K = 4096
_W = 64
_ROWS = 512
_NSTEP = _N0 // _ROWS
_PADK = 1024


def _reduce_body(x_ref, mv_ref, mi_ref):
    x = x_ref[...]
    ntile = _K // 128
    vm = x[:, :, 0:128]
    it = jnp.zeros((1, _ROWS, 128), jnp.int32)
    for t in range(1, ntile):
        xt = x[:, :, t * 128:(t + 1) * 128]
        gt = xt > vm
        it = jnp.where(gt, t, it)
        vm = jnp.where(gt, xt, vm)
    m = jnp.max(vm, axis=-1)
    lane = lax.broadcasted_iota(jnp.int32, (1, _ROWS, 128), 2)
    g = (it << 7) | lane
    hit = jnp.where(vm == m[..., None], g, _K)
    mi = jnp.min(hit, axis=-1)
    mv_ref[...] = m[:, None, :]
    mi_ref[...] = mi[:, None, :]


def _partner(x, j, n):
    fwd = jnp.concatenate([x[:, j:], x[:, :j]], axis=1)
    bwd = jnp.concatenate([x[:, n - j:], x[:, :n - j]], axis=1)
    return fwd, bwd


def _topk_body(mv_ref, mi_ref, conf_ref, x0_ref, y0_ref, x1_ref, y1_ref):
    v = mv_ref[...]
    anch = mi_ref[...]
    v = jnp.where(v > _CONF, v, -jnp.inf)
    idx = lax.broadcasted_iota(jnp.int32, (_B, _N0), 1)
    pack = (idx << 12) | anch

    n = _N0
    k = 2
    while k <= n:
        j = k // 2
        while j >= 1:
            bit_lo = (idx & j) == 0
            desc = (idx & k) == 0
            vf, vb = _partner(v, j, n)
            pf, pb = _partner(pack, j, n)
            pv = jnp.where(bit_lo, vf, vb)
            pp = jnp.where(bit_lo, pf, pb)
            self_wins = (v > pv) | ((v == pv) & (pack < pp))
            keep = (self_wins == bit_lo) == desc
            v = jnp.where(keep, v, pv)
            pack = jnp.where(keep, pack, pp)
            j //= 2
        k *= 2

    top = v[:, :_PADK]
    pk = pack[:, :_PADK]
    sidx = pk >> 12
    sanch = pk & (_N0 - 1)
    valid = top > _CONF
    inv = jnp.float32(1.0 / (_W - 1))
    fz = jnp.float32(0.0)
    conf_ref[...] = jnp.where(valid, top, fz)
    x0_ref[...] = jnp.where(valid, (sidx & (_W - 1)).astype(jnp.float32) * inv, fz)
    y0_ref[...] = jnp.where(valid, (sidx >> 6).astype(jnp.float32) * inv, fz)
    x1_ref[...] = jnp.where(valid, (sanch & (_W - 1)).astype(jnp.float32) * inv, fz)
    y1_ref[...] = jnp.where(valid, (sanch >> 6).astype(jnp.float32) * inv, fz)


def kernel(anchor_probs):
    B, N0, K = anchor_probs.shape
    mv3, mi3 = pl.pallas_call(
        _reduce_body,
        grid=(B * _NSTEP,),
        in_specs=[pl.BlockSpec((1, _ROWS, K), lambda g: (g // _NSTEP, g % _NSTEP, 0))],
        out_specs=[
            pl.BlockSpec((1, 1, _ROWS), lambda g: (g, 0, 0)),
            pl.BlockSpec((1, 1, _ROWS), lambda g: (g, 0, 0)),
        ],
        out_shape=[
            jax.ShapeDtypeStruct((B * _NSTEP, 1, _ROWS), jnp.float32),
            jax.ShapeDtypeStruct((B * _NSTEP, 1, _ROWS), jnp.int32),
        ],
    )(anchor_probs)
    maxv = mv3.reshape(B, N0)
    maxi = mi3.reshape(B, N0)

    conf, x0, y0, x1, y1 = pl.pallas_call(
        _topk_body,
        out_shape=[jax.ShapeDtypeStruct((B, _PADK), jnp.float32)] * 5,
    )(maxv, maxi)

    conf = conf[:, :_TOP_K]
    mkpts0 = jnp.stack([x0[:, :_TOP_K], y0[:, :_TOP_K]], axis=-1).reshape(-1, 2)
    mkpts1 = jnp.stack([x1[:, :_TOP_K], y1[:, :_TOP_K]], axis=-1).reshape(-1, 2)
    mconf = conf.reshape(-1)
    b_ids = jnp.broadcast_to(jnp.arange(B)[:, None], (B, _TOP_K)).reshape(-1)
    return (mkpts0, mkpts1, mconf, b_ids)
